# Initial kernel scaffold; baseline (speedup 1.0000x reference)
import jax
import jax.numpy as jnp
from jax.experimental import pallas as pl


def kernel(rpn_score, anchors):
    raise NotImplementedError("write your pallas kernel here")


# speedup vs baseline: 6.1186x; 6.1186x over previous
import functools

import jax
import jax.numpy as jnp
from jax.experimental import pallas as pl

_PROPOSAL_NUM = 6
_IOU_THR = 0.25
_LANES = 128


def _nms_block_kernel(scores_ref, anchors_ref, out_ref, *, n_real, k, iou_thr):
    s = scores_ref[...]
    y1 = anchors_ref[0:1, :]
    x1 = anchors_ref[1:2, :]
    y2 = anchors_ref[2:3, :]
    x2 = anchors_ref[3:4, :]
    areas = (y2 - y1) * (x2 - x1)

    br, np_ = s.shape
    iota = jax.lax.broadcasted_iota(jnp.int32, (1, np_), 1)
    valid = jnp.broadcast_to(iota < n_real, s.shape)

    neg_inf = jnp.float32(-jnp.inf)
    cols = []
    for _ in range(k):
        masked = jnp.where(valid, s, neg_inf)
        m = jnp.max(masked, axis=1, keepdims=True)
        eq = masked == m
        idx = jnp.min(jnp.where(eq, iota, np_), axis=1, keepdims=True)
        sel = (iota == idx).astype(jnp.float32)
        by1 = jnp.sum(sel * y1, axis=1, keepdims=True)
        bx1 = jnp.sum(sel * x1, axis=1, keepdims=True)
        by2 = jnp.sum(sel * y2, axis=1, keepdims=True)
        bx2 = jnp.sum(sel * x2, axis=1, keepdims=True)
        barea = (by2 - by1) * (bx2 - bx1)
        yy1 = jnp.maximum(by1, y1)
        xx1 = jnp.maximum(bx1, x1)
        yy2 = jnp.minimum(by2, y2)
        xx2 = jnp.minimum(bx2, x2)
        inter = jnp.maximum(yy2 - yy1, 0.0) * jnp.maximum(xx2 - xx1, 0.0)
        iou = inter / (barea + areas - inter + 1e-9)
        valid = valid & (iou <= iou_thr) & (iota != idx)
        cols.append(idx)

    out_ref[...] = jnp.concatenate(cols, axis=1)


def kernel(rpn_score, anchors):
    b, n = rpn_score.shape
    np_ = ((n + _LANES - 1) // _LANES) * _LANES
    pad = np_ - n
    scores = jnp.pad(rpn_score, ((0, 0), (0, pad)),
                     constant_values=-jnp.inf)
    anchors_t = jnp.pad(anchors.T, ((0, 0), (0, pad)))

    block_rows = 8
    grid = (b // block_rows,)
    body = functools.partial(_nms_block_kernel, n_real=n,
                             k=_PROPOSAL_NUM, iou_thr=_IOU_THR)
    out = pl.pallas_call(
        body,
        grid=grid,
        in_specs=[
            pl.BlockSpec((block_rows, np_), lambda i: (i, 0)),
            pl.BlockSpec((4, np_), lambda i: (0, 0)),
        ],
        out_specs=pl.BlockSpec((block_rows, _PROPOSAL_NUM), lambda i: (i, 0)),
        out_shape=jax.ShapeDtypeStruct((b, _PROPOSAL_NUM), jnp.int32),
    )(scores, anchors_t)
    return out


# speedup vs baseline: 6.2602x; 1.0231x over previous
import functools

import jax
import jax.numpy as jnp
from jax.experimental import pallas as pl
from jax.experimental.pallas import tpu as pltpu

_PROPOSAL_NUM = 6
_IOU_THR = 0.25
_LANES = 128


def _nms_block_kernel(scores_ref, anchors_ref, out_ref, *, n_real, k, iou_thr):
    del n_real
    ms = scores_ref[...]
    y1 = anchors_ref[0:1, :]
    x1 = anchors_ref[1:2, :]
    y2 = anchors_ref[2:3, :]
    x2 = anchors_ref[3:4, :]
    areas = (y2 - y1) * (x2 - x1)

    br, np_ = ms.shape
    iota = jax.lax.broadcasted_iota(jnp.int32, (1, np_), 1)

    neg_inf = jnp.float32(-jnp.inf)
    cols = []
    for step in range(k):
        m = jnp.max(ms, axis=1, keepdims=True)
        eq = ms == m
        idx = jnp.min(jnp.where(eq, iota, np_), axis=1, keepdims=True)
        cols.append(idx)
        if step == k - 1:
            break
        sel = (iota == idx).astype(jnp.float32)
        by1 = jnp.sum(sel * y1, axis=1, keepdims=True)
        bx1 = jnp.sum(sel * x1, axis=1, keepdims=True)
        by2 = jnp.sum(sel * y2, axis=1, keepdims=True)
        bx2 = jnp.sum(sel * x2, axis=1, keepdims=True)
        barea = (by2 - by1) * (bx2 - bx1)
        yy1 = jnp.maximum(by1, y1)
        xx1 = jnp.maximum(bx1, x1)
        yy2 = jnp.minimum(by2, y2)
        xx2 = jnp.minimum(bx2, x2)
        inter = jnp.maximum(yy2 - yy1, 0.0) * jnp.maximum(xx2 - xx1, 0.0)
        iou = inter / (barea + areas - inter + 1e-9)
        ms = jnp.where((iou <= iou_thr) & (iota != idx), ms, neg_inf)

    out_ref[...] = jnp.concatenate(cols, axis=1)


def kernel(rpn_score, anchors):
    b, n = rpn_score.shape
    np_ = ((n + _LANES - 1) // _LANES) * _LANES
    pad = np_ - n
    scores = jnp.pad(rpn_score, ((0, 0), (0, pad)),
                     constant_values=-jnp.inf)
    anchors_t = jnp.pad(anchors.T, ((0, 0), (0, pad)))

    block_rows = 8
    grid = (b // block_rows,)
    body = functools.partial(_nms_block_kernel, n_real=n,
                             k=_PROPOSAL_NUM, iou_thr=_IOU_THR)
    out = pl.pallas_call(
        body,
        grid=grid,
        in_specs=[
            pl.BlockSpec((block_rows, np_), lambda i: (i, 0)),
            pl.BlockSpec((4, np_), lambda i: (0, 0)),
        ],
        out_specs=pl.BlockSpec((block_rows, _PROPOSAL_NUM), lambda i: (i, 0)),
        out_shape=jax.ShapeDtypeStruct((b, _PROPOSAL_NUM), jnp.int32),
        compiler_params=pltpu.CompilerParams(
            dimension_semantics=("parallel",)),
    )(scores, anchors_t)
    return out
